# Initial kernel scaffold; baseline (speedup 1.0000x reference)
import functools

import jax
import jax.numpy as jnp
from jax import lax
from jax.experimental import pallas as pl
from jax.experimental.pallas import tpu as pltpu
from jax.experimental.pallas import tpu_sc as plsc

N = 10000
D = 256
DH = 128
E = 160000
NCORE = 2
NSUB = 16
EPT = E // NSUB
K = 80
NCHUNK = EPT // K
RPT = N // NSUB
RCHUNK = 125
NRC = RPT // RCHUNK
ROWB = 1000


def _mm_body(x_ref, w_ref, o_ref):
    o_ref[0] = lax.dot_general(
        x_ref[...], w_ref[...], (((1,), (1,)), ((), ())),
        preferred_element_type=jnp.float32)


def _xw_split(x, W):
    return pl.pallas_call(
        _mm_body,
        grid=(NCORE, N // ROWB),
        in_specs=[pl.BlockSpec((ROWB, D), lambda c, r: (r, 0)),
                  pl.BlockSpec((DH, D), lambda c, r: (c, 0))],
        out_specs=pl.BlockSpec((1, ROWB, DH), lambda c, r: (c, r, 0)),
        out_shape=jax.ShapeDtypeStruct((NCORE, N, DH), jnp.float32),
    )(x, W)


def _sc_scatter(xw_flat, row_idx, col_idx, zeros128, zeros16, ones16):
    mesh = plsc.VectorSubcoreMesh(core_axis_name="c", subcore_axis_name="s")

    @functools.partial(
        pl.kernel,
        out_type=[jax.ShapeDtypeStruct((NCORE, N, DH), jnp.float32),
                  jax.ShapeDtypeStruct((N, 16), jnp.float32)],
        mesh=mesh,
        scratch_types=[
            pltpu.VMEM_SHARED((N, DH), jnp.float32),
            pltpu.VMEM_SHARED((N, 16), jnp.float32),
            pltpu.VMEM((NCHUNK, K), jnp.int32),
            pltpu.VMEM((NCHUNK, K), jnp.int32),
            pltpu.VMEM((K, DH), jnp.float32),
            pltpu.VMEM((K, 16), jnp.float32),
            pltpu.SemaphoreType.DMA,
        ],
    )
    def k(xw_hbm, row_hbm, col_hbm, z128_hbm, z16_hbm, ones_hbm,
          out_hbm, deg_hbm, acc, dacc, rbuf, cbuf, gbuf, obuf, sem):
        c = lax.axis_index("c")
        s = lax.axis_index("s")
        r0 = s * RPT
        is0 = c == 0
        for t in range(NRC):
            pltpu.sync_copy(z128_hbm, acc.at[pl.ds(r0 + t * RCHUNK, RCHUNK)])

        @pl.when(is0)
        def _():
            pltpu.sync_copy(z16_hbm, dacc.at[pl.ds(r0, RPT)])

        pltpu.sync_copy(ones_hbm, obuf)
        pltpu.sync_copy(row_hbm.at[s], rbuf)
        pltpu.sync_copy(col_hbm.at[c, s], cbuf)
        plsc.subcore_barrier()

        def body(j, carry):
            pltpu.async_copy(xw_hbm.at[cbuf.at[j]], gbuf, sem).wait()
            pltpu.sync_copy(gbuf, acc.at[rbuf.at[j]], add=True)

            @pl.when(is0)
            def _():
                pltpu.sync_copy(obuf, dacc.at[rbuf.at[j]], add=True)

            return carry

        lax.fori_loop(0, NCHUNK, body, 0)
        plsc.subcore_barrier()
        for t in range(NRC):
            sl = pl.ds(r0 + t * RCHUNK, RCHUNK)
            pltpu.sync_copy(acc.at[sl], out_hbm.at[c].at[sl])

        @pl.when(is0)
        def _():
            pltpu.sync_copy(dacc.at[pl.ds(r0, RPT)], deg_hbm.at[pl.ds(r0, RPT)])

    return k(xw_flat, row_idx, col_idx, zeros128, zeros16, ones16)


def _finish_body(a_ref, d_ref, b_ref, o_ref):
    d = jnp.maximum(d_ref[:, :1], 1.0)
    o_ref[:, :DH] = a_ref[0] / d + b_ref[0, :DH]
    o_ref[:, DH:] = a_ref[1] / d + b_ref[0, DH:]


def _finish(acc2, deg16, b2):
    return pl.pallas_call(
        _finish_body,
        grid=(N // ROWB,),
        in_specs=[pl.BlockSpec((NCORE, ROWB, DH), lambda r: (0, r, 0)),
                  pl.BlockSpec((ROWB, 16), lambda r: (r, 0)),
                  pl.BlockSpec((1, D), lambda r: (0, 0))],
        out_specs=pl.BlockSpec((ROWB, D), lambda r: (r, 0)),
        out_shape=jax.ShapeDtypeStruct((N, D), jnp.float32),
    )(acc2, deg16, b2)


def kernel(x, edge_index, W, b):
    row = edge_index[0].astype(jnp.int32)
    col = edge_index[1].astype(jnp.int32)
    xw = _xw_split(x, W)
    xw_flat = xw.reshape(NCORE * N, DH)
    row_r = row.reshape(NSUB, NCHUNK, K)
    col_adj = jnp.stack([col, col + N]).reshape(NCORE, NSUB, NCHUNK, K)
    zeros128 = jnp.zeros((RCHUNK, DH), jnp.float32)
    zeros16 = jnp.zeros((RPT, 16), jnp.float32)
    ones16 = jnp.ones((K, 16), jnp.float32)
    acc, deg = _sc_scatter(xw_flat, row_r, col_adj, zeros128, zeros16, ones16)
    return _finish(acc, deg, b.reshape(1, D))


# speedup vs baseline: 3.8812x; 3.8812x over previous
import functools

import jax
import jax.numpy as jnp
from jax import lax
from jax.experimental import pallas as pl
from jax.experimental.pallas import tpu as pltpu
from jax.experimental.pallas import tpu_sc as plsc

N = 10000
D = 256
DH = 128
E = 160000
NCORE = 2
NSUB = 16
EPT = E // NSUB
K = 128
NCHUNK = 79
EPTP = NCHUNK * K
NP = 10240
DUMMY = NP - 4
RPT = NP // NSUB
RCHUNK = 128
NRC = RPT // RCHUNK
ROWB = 1000


def _mm_body(x_ref, w_ref, o_ref):
    o_ref[0] = lax.dot_general(
        x_ref[...], w_ref[...], (((1,), (1,)), ((), ())),
        preferred_element_type=jnp.float32)


def _xw_split(x, W):
    return pl.pallas_call(
        _mm_body,
        grid=(NCORE, N // ROWB),
        in_specs=[pl.BlockSpec((ROWB, D), lambda c, r: (r, 0)),
                  pl.BlockSpec((DH, D), lambda c, r: (c, 0))],
        out_specs=pl.BlockSpec((1, ROWB, DH), lambda c, r: (c, r, 0)),
        out_shape=jax.ShapeDtypeStruct((NCORE, N, DH), jnp.float32),
    )(x, W)


def _sc_scatter(xw_flat, row_idx, col_idx):
    mesh = plsc.VectorSubcoreMesh(core_axis_name="c", subcore_axis_name="s")

    @functools.partial(
        pl.kernel,
        out_type=[jax.ShapeDtypeStruct((NCORE, NP, DH), jnp.float32),
                  jax.ShapeDtypeStruct((NP,), jnp.float32)],
        mesh=mesh,
        scratch_types=[
            pltpu.VMEM_SHARED((NP, DH), jnp.float32),
            pltpu.VMEM_SHARED((NP,), jnp.float32),
            pltpu.VMEM((NCHUNK, K), jnp.int32),
            pltpu.VMEM((NCHUNK, K), jnp.int32),
            pltpu.VMEM((K, DH), jnp.float32),
            pltpu.VMEM((K,), jnp.float32),
            pltpu.SemaphoreType.DMA,
        ],
    )
    def k(xw_hbm, row_hbm, col_hbm,
          out_hbm, deg_hbm, acc, dacc, rbuf, cbuf, gbuf, obuf, sem):
        c = lax.axis_index("c")
        s = lax.axis_index("s")
        r0 = s * RPT
        is0 = c == 0

        z16 = jnp.zeros((16,), jnp.float32)

        def zfill(j, carry):
            gbuf[j // 8, pl.ds((j % 8) * 16, 16)] = z16
            return carry

        lax.fori_loop(0, K * 8, zfill, 0)
        for j in range(K // 16):
            obuf[pl.ds(j * 16, 16)] = z16
        for t in range(NRC):
            pltpu.sync_copy(gbuf, acc.at[pl.ds(r0 + t * RCHUNK, RCHUNK)])
        pltpu.sync_copy(gbuf.at[0], dacc.at[pl.ds(r0, RCHUNK)])
        for t in range(1, NRC):
            pltpu.sync_copy(gbuf.at[0], dacc.at[pl.ds(r0 + t * RCHUNK, RCHUNK)])
        o16 = jnp.ones((16,), jnp.float32)
        for j in range(K // 16):
            obuf[pl.ds(j * 16, 16)] = o16
        pltpu.sync_copy(row_hbm.at[s], rbuf)
        pltpu.sync_copy(col_hbm.at[c, s], cbuf)
        plsc.subcore_barrier()

        def body(j, carry):
            pltpu.async_copy(xw_hbm.at[cbuf.at[j]], gbuf, sem).wait()
            pltpu.sync_copy(gbuf, acc.at[rbuf.at[j]], add=True)

            @pl.when(is0)
            def _():
                pltpu.sync_copy(obuf, dacc.at[rbuf.at[j]], add=True)

            return carry

        lax.fori_loop(0, NCHUNK, body, 0)
        plsc.subcore_barrier()
        for t in range(NRC):
            sl = pl.ds(r0 + t * RCHUNK, RCHUNK)
            pltpu.sync_copy(acc.at[sl], out_hbm.at[c].at[sl])

        @pl.when(is0)
        def _():
            pltpu.sync_copy(dacc.at[pl.ds(r0, RPT)], deg_hbm.at[pl.ds(r0, RPT)])

    return k(xw_flat, row_idx, col_idx)


def _finish_body(a_ref, d_ref, b_ref, o_ref):
    d = jnp.maximum(d_ref[...], 1.0)
    o_ref[:, :DH] = a_ref[0] / d + b_ref[0, :DH]
    o_ref[:, DH:] = a_ref[1] / d + b_ref[0, DH:]


def _finish(acc2, deg2, b2):
    return pl.pallas_call(
        _finish_body,
        grid=(N // ROWB,),
        in_specs=[pl.BlockSpec((NCORE, ROWB, DH), lambda r: (0, r, 0)),
                  pl.BlockSpec((ROWB, 1), lambda r: (r, 0)),
                  pl.BlockSpec((1, D), lambda r: (0, 0))],
        out_specs=pl.BlockSpec((ROWB, D), lambda r: (r, 0)),
        out_shape=jax.ShapeDtypeStruct((N, D), jnp.float32),
    )(acc2, deg2, b2)


def kernel(x, edge_index, W, b):
    row = edge_index[0].astype(jnp.int32)
    col = edge_index[1].astype(jnp.int32)
    xw = _xw_split(x, W)
    xw_flat = xw.reshape(NCORE * N, DH)
    pad = ((0, 0), (0, EPTP - EPT))
    row_r = jnp.pad(row.reshape(NSUB, EPT), pad,
                    constant_values=DUMMY).reshape(NSUB, NCHUNK, K)
    col_p = jnp.pad(col.reshape(NSUB, EPT), pad, constant_values=0)
    col_adj = jnp.stack([col_p, col_p + N]).reshape(NCORE, NSUB, NCHUNK, K)
    acc, deg = _sc_scatter(xw_flat, row_r, col_adj)
    return _finish(acc, deg[:N].reshape(N, 1), b.reshape(1, D))


# speedup vs baseline: 5.0557x; 1.3026x over previous
import functools

import jax
import jax.numpy as jnp
from jax import lax
from jax.experimental import pallas as pl
from jax.experimental.pallas import tpu as pltpu
from jax.experimental.pallas import tpu_sc as plsc

N = 10000
D = 256
DH = 128
E = 160000
NCORE = 2
NSUB = 16
EPT = E // NSUB
K = 128
NCHUNK = 79
EPTP = NCHUNK * K
NP = 10240
DUMMY = NP - 4
RPT = NP // NSUB
RCHUNK = 128
NRC = RPT // RCHUNK
ROWB = 1000


def _mm_body(x_ref, w_ref, o_ref):
    o_ref[0] = lax.dot_general(
        x_ref[...], w_ref[...], (((1,), (1,)), ((), ())),
        preferred_element_type=jnp.float32)


def _xw_split(x, W):
    return pl.pallas_call(
        _mm_body,
        grid=(NCORE, N // ROWB),
        in_specs=[pl.BlockSpec((ROWB, D), lambda c, r: (r, 0)),
                  pl.BlockSpec((DH, D), lambda c, r: (c, 0))],
        out_specs=pl.BlockSpec((1, ROWB, DH), lambda c, r: (c, r, 0)),
        out_shape=jax.ShapeDtypeStruct((NCORE, N, DH), jnp.float32),
    )(x, W)


def _sc_scatter(xw_flat, packed_idx):
    mesh = plsc.VectorSubcoreMesh(core_axis_name="c", subcore_axis_name="s")

    @functools.partial(
        pl.kernel,
        out_type=[jax.ShapeDtypeStruct((NCORE, NP, DH), jnp.float32),
                  jax.ShapeDtypeStruct((NP,), jnp.float32)],
        mesh=mesh,
        scratch_types=[
            pltpu.VMEM_SHARED((NP, DH), jnp.float32),
            pltpu.VMEM_SHARED((NP,), jnp.float32),
            pltpu.VMEM((NCHUNK, K), jnp.int32),
            pltpu.VMEM((K, DH), jnp.float32),
            pltpu.VMEM((K, DH), jnp.float32),
            pltpu.VMEM((K,), jnp.int32),
            pltpu.VMEM((K,), jnp.int32),
            pltpu.VMEM((K,), jnp.int32),
            pltpu.VMEM((K,), jnp.int32),
            pltpu.VMEM((K,), jnp.float32),
            pltpu.SemaphoreType.DMA,
            pltpu.SemaphoreType.DMA,
        ],
    )
    def k(xw_hbm, pidx_hbm, out_hbm, deg_hbm, acc, dacc, pbuf,
          gbufA, gbufB, cbA, rbA, cbB, rbB, obuf, semA, semB):
        c = lax.axis_index("c")
        s = lax.axis_index("s")
        r0 = s * RPT
        is0 = c == 0

        z16 = jnp.zeros((16,), jnp.float32)

        def zfill(j, carry):
            gbufA[j // 8, pl.ds((j % 8) * 16, 16)] = z16
            return carry

        lax.fori_loop(0, K * 8, zfill, 0)
        for j in range(K // 16):
            obuf[pl.ds(j * 16, 16)] = z16
        for t in range(NRC):
            pltpu.sync_copy(gbufA, acc.at[pl.ds(r0 + t * RCHUNK, RCHUNK)])
        for t in range(NRC):
            pltpu.sync_copy(gbufA.at[0], dacc.at[pl.ds(r0 + t * RCHUNK, RCHUNK)])
        o16 = jnp.ones((16,), jnp.float32)
        for j in range(K // 16):
            obuf[pl.ds(j * 16, 16)] = o16
        pltpu.sync_copy(pidx_hbm.at[c, s], pbuf)

        def unpack(j, cb, rb):
            for t in range(K // 16):
                sl = pl.ds(t * 16, 16)
                p = pbuf[j, sl]
                cb[sl] = p & 0x7FFF
                rb[sl] = lax.shift_right_logical(p, 15)

        plsc.subcore_barrier()

        unpack(0, cbA, rbA)
        pltpu.make_async_copy(xw_hbm.at[cbA], gbufA, semA).start()
        unpack(1, cbB, rbB)

        def body(i, carry):
            j0 = 2 * i
            pltpu.make_async_copy(xw_hbm.at[cbB], gbufB, semB).start()
            pltpu.make_async_copy(xw_hbm.at[cbA], gbufA, semA).wait()
            pltpu.sync_copy(gbufA, acc.at[rbA], add=True)

            @pl.when(is0)
            def _():
                pltpu.sync_copy(obuf, dacc.at[rbA], add=True)

            unpack(j0 + 2, cbA, rbA)
            pltpu.make_async_copy(xw_hbm.at[cbA], gbufA, semA).start()
            pltpu.make_async_copy(xw_hbm.at[cbB], gbufB, semB).wait()
            pltpu.sync_copy(gbufB, acc.at[rbB], add=True)

            @pl.when(is0)
            def _():
                pltpu.sync_copy(obuf, dacc.at[rbB], add=True)

            @pl.when(j0 + 3 < NCHUNK)
            def _():
                unpack(j0 + 3, cbB, rbB)

            return carry

        lax.fori_loop(0, (NCHUNK - 1) // 2, body, 0)
        pltpu.make_async_copy(xw_hbm.at[cbA], gbufA, semA).wait()
        pltpu.sync_copy(gbufA, acc.at[rbA], add=True)

        @pl.when(is0)
        def _():
            pltpu.sync_copy(obuf, dacc.at[rbA], add=True)

        plsc.subcore_barrier()
        for t in range(NRC):
            sl = pl.ds(r0 + t * RCHUNK, RCHUNK)
            pltpu.sync_copy(acc.at[sl], out_hbm.at[c].at[sl])

        @pl.when(is0)
        def _():
            pltpu.sync_copy(dacc.at[pl.ds(r0, RPT)], deg_hbm.at[pl.ds(r0, RPT)])

    return k(xw_flat, packed_idx)


def _finish_body(a_ref, d_ref, b_ref, o_ref):
    d = jnp.maximum(d_ref[...], 1.0)
    o_ref[:, :DH] = a_ref[0] / d + b_ref[0, :DH]
    o_ref[:, DH:] = a_ref[1] / d + b_ref[0, DH:]


def _finish(acc2, deg2, b2):
    return pl.pallas_call(
        _finish_body,
        grid=(N // ROWB,),
        in_specs=[pl.BlockSpec((NCORE, ROWB, DH), lambda r: (0, r, 0)),
                  pl.BlockSpec((ROWB, 1), lambda r: (r, 0)),
                  pl.BlockSpec((1, D), lambda r: (0, 0))],
        out_specs=pl.BlockSpec((ROWB, D), lambda r: (r, 0)),
        out_shape=jax.ShapeDtypeStruct((N, D), jnp.float32),
    )(acc2, deg2, b2)


def kernel(x, edge_index, W, b):
    row = edge_index[0].astype(jnp.int32)
    col = edge_index[1].astype(jnp.int32)
    xw = _xw_split(x, W)
    xw_flat = xw.reshape(NCORE * N, DH)
    pad = ((0, 0), (0, EPTP - EPT))
    row_p = jnp.pad(row.reshape(NSUB, EPT), pad, constant_values=DUMMY)
    col_p = jnp.pad(col.reshape(NSUB, EPT), pad, constant_values=0)
    packed = jnp.stack([col_p, col_p + N]) | (row_p << 15)[None]
    acc, deg = _sc_scatter(xw_flat, packed.reshape(NCORE, NSUB, NCHUNK, K))
    return _finish(acc, deg[:N].reshape(N, 1), b.reshape(1, D))
